# baseline (speedup 1.0000x reference)
import functools

import jax
import jax.numpy as jnp
from jax import lax
from jax.experimental import pallas as pl
from jax.experimental.pallas import tpu as pltpu
from jax.experimental.pallas import tpu_sc as plsc

B = 16384
EMBED_DIM = 16
NC = 2
NS = 16
NW = NC * NS
B_PER_W = B // NW


def _sc_gather_body(uidx_hbm, iidx_hbm, utab_hbm, itab_hbm,
                    uout_hbm, iout_hbm,
                    idx_u, idx_i, rows_u, rows_i, sem_u, sem_i):
    wid = lax.axis_index("s") * NC + lax.axis_index("c")
    base = wid * B_PER_W
    pltpu.sync_copy(uidx_hbm.at[pl.ds(base, B_PER_W)], idx_u)
    pltpu.sync_copy(iidx_hbm.at[pl.ds(base, B_PER_W)], idx_i)
    cu = pltpu.async_copy(utab_hbm.at[idx_u], rows_u, sem_u)
    ci = pltpu.async_copy(itab_hbm.at[idx_i], rows_i, sem_i)
    cu.wait()
    ci.wait()
    pltpu.sync_copy(rows_u, uout_hbm.at[pl.ds(base, B_PER_W)])
    pltpu.sync_copy(rows_i, iout_hbm.at[pl.ds(base, B_PER_W)])


@jax.jit
def _sc_gather(user_idx, item_idx, user_table, item_table):
    mesh = plsc.VectorSubcoreMesh(core_axis_name="c", subcore_axis_name="s")
    f = pl.kernel(
        _sc_gather_body,
        out_type=(
            jax.ShapeDtypeStruct((B, EMBED_DIM), jnp.float32),
            jax.ShapeDtypeStruct((B, EMBED_DIM), jnp.float32),
        ),
        mesh=mesh,
        scratch_types=[
            pltpu.VMEM((B_PER_W,), jnp.int32),
            pltpu.VMEM((B_PER_W,), jnp.int32),
            pltpu.VMEM((B_PER_W, EMBED_DIM), jnp.float32),
            pltpu.VMEM((B_PER_W, EMBED_DIM), jnp.float32),
            pltpu.SemaphoreType.DMA,
            pltpu.SemaphoreType.DMA,
        ],
        compiler_params=pltpu.CompilerParams(use_tc_tiling_on_sc=False),
    )
    return f(user_idx, item_idx, user_table, item_table)


def _mlp_body(u_ref, i_ref, dc_ref, w1u_ref, w1i_ref, w1dc_ref, b1_ref,
              w2_ref, b2_ref, wh_ref, bh_ref, out_ref):
    h = (jnp.dot(u_ref[...], w1u_ref[...], preferred_element_type=jnp.float32)
         + jnp.dot(i_ref[...], w1i_ref[...], preferred_element_type=jnp.float32)
         + jnp.dot(dc_ref[...], w1dc_ref[...], preferred_element_type=jnp.float32)
         + b1_ref[...])
    h = jnp.maximum(h, 0.0)
    h = jnp.dot(h, w2_ref[...], preferred_element_type=jnp.float32) + b2_ref[...]
    h = jnp.maximum(h, 0.0)
    out_ref[...] = jax.nn.sigmoid(
        jnp.dot(h, wh_ref[...], preferred_element_type=jnp.float32) + bh_ref[...])


@jax.jit
def _tc_mlp(u_emb, i_emb, dc, w1u, w1i, w1dc, b1, w2, b2, wh, bh):
    BB = 2048
    grid = (B // BB,)
    dcdim = dc.shape[1]
    return pl.pallas_call(
        _mlp_body,
        grid=grid,
        in_specs=[
            pl.BlockSpec((BB, EMBED_DIM), lambda i: (i, 0)),
            pl.BlockSpec((BB, EMBED_DIM), lambda i: (i, 0)),
            pl.BlockSpec((BB, dcdim), lambda i: (i, 0)),
            pl.BlockSpec(w1u.shape, lambda i: (0, 0)),
            pl.BlockSpec(w1i.shape, lambda i: (0, 0)),
            pl.BlockSpec(w1dc.shape, lambda i: (0, 0)),
            pl.BlockSpec(b1.shape, lambda i: (0, 0)),
            pl.BlockSpec(w2.shape, lambda i: (0, 0)),
            pl.BlockSpec(b2.shape, lambda i: (0, 0)),
            pl.BlockSpec(wh.shape, lambda i: (0, 0)),
            pl.BlockSpec(bh.shape, lambda i: (0, 0)),
        ],
        out_specs=pl.BlockSpec((BB, 2), lambda i: (i, 0)),
        out_shape=jax.ShapeDtypeStruct((B, 2), jnp.float32),
    )(u_emb, i_emb, dc, w1u, w1i, w1dc, b1, w2, b2, wh, bh)


def kernel(user_idx, item_idx, dense_feats, comment_emb, user_table, item_table,
           W1, b1, W2, b2, ctr_w, ctr_b, cvr_w, cvr_b):
    user_idx = user_idx.astype(jnp.int32)
    item_idx = item_idx.astype(jnp.int32)
    u_emb, i_emb = _sc_gather(user_idx, item_idx, user_table, item_table)
    dc = jnp.concatenate([dense_feats, comment_emb], axis=-1)
    w1u = W1[:EMBED_DIM]
    w1i = W1[EMBED_DIM:2 * EMBED_DIM]
    w1dc = W1[2 * EMBED_DIM:]
    wh = jnp.concatenate([ctr_w, cvr_w], axis=1)
    bh = jnp.stack([ctr_b[0], cvr_b[0]])[None, :]
    out = _tc_mlp(u_emb, i_emb, dc, w1u, w1i, w1dc, b1[None, :], W2,
                  b2[None, :], wh, bh)
    return out[:, 0], out[:, 1]
